# baseline (speedup 1.0000x reference)
import functools
import math

import jax
import jax.numpy as jnp
from jax import lax
from jax.experimental import pallas as pl
from jax.experimental.pallas import tpu as pltpu
from jax.experimental.pallas import tpu_sc as plsc

N_NODES = 50000
N_EDGES = 800000
N_TRIPLETS = 800000
NUM_RADIAL = 6
NUM_SPHERICAL = 7
HIDDEN = 128
CUTOFF = 5.0

SC_CORES = 2
SC_SUBCORES = 16
NWORK = SC_CORES * SC_SUBCORES
CHUNK = 2000
NCHUNK = N_TRIPLETS // CHUNK
GMAX = (NCHUNK + NWORK - 1) // NWORK
LANES = 16

BT = 8000
NBT = N_TRIPLETS // BT

SLICES = 4
SLEN = N_TRIPLETS // SLICES
S_NCHUNK = SLEN // CHUNK
S_GMAX = (S_NCHUNK + NWORK - 1) // NWORK
NBS = SLEN // BT


def _edge_body(ei0_hbm, ei1_hbm, posx_hbm, posy_hbm, posz_hbm,
               d2e_hbm,
               sposx, sposy, sposz,
               ba0, ba1, gxa, gya, gza, gxb, gyb, gzb, od2):
    cid = lax.axis_index("c")
    sid = lax.axis_index("s")
    wid = sid * SC_CORES + cid

    @pl.when(sid == 0)
    def _():
        pltpu.sync_copy(posx_hbm, sposx)
        pltpu.sync_copy(posy_hbm, sposy)
        pltpu.sync_copy(posz_hbm, sposz)

    plsc.subcore_barrier()

    @pl.loop(0, GMAX)
    def _(g):
        chunk = wid + g * NWORK

        @pl.when(chunk < NCHUNK)
        def _():
            sl = pl.ds(chunk * CHUNK, CHUNK)
            pltpu.sync_copy(ei0_hbm.at[sl], ba0)
            pltpu.sync_copy(ei1_hbm.at[sl], ba1)
            pltpu.sync_copy(sposx.at[ba0], gxa)
            pltpu.sync_copy(sposy.at[ba0], gya)
            pltpu.sync_copy(sposz.at[ba0], gza)
            pltpu.sync_copy(sposx.at[ba1], gxb)
            pltpu.sync_copy(sposy.at[ba1], gyb)
            pltpu.sync_copy(sposz.at[ba1], gzb)

            @pl.loop(0, CHUNK, step=LANES)
            def _(v):
                s = pl.ds(v, LANES)
                dx = gxa[s] - gxb[s]
                dy = gya[s] - gyb[s]
                dz = gza[s] - gzb[s]
                od2[s] = dx * dx + dy * dy + dz * dz

            pltpu.sync_copy(od2, d2e_hbm.at[sl])


def _triplet_body(i_hbm, j_hbm, k_hbm, e_hbm,
                  posx_hbm, posy_hbm, posz_hbm, d2e_hbm,
                  xdot_hbm, y2_hbm, d2_hbm,
                  sposx, sposy, sposz, sd2e,
                  bi, bj, bk, be,
                  gxi, gyi, gzi, gxj, gyj, gzj, gxk, gyk, gzk,
                  gd2, oxdot, oy2):
    cid = lax.axis_index("c")
    sid = lax.axis_index("s")
    wid = sid * SC_CORES + cid

    @pl.when(sid == 0)
    def _():
        pltpu.sync_copy(posx_hbm, sposx)
        pltpu.sync_copy(posy_hbm, sposy)
        pltpu.sync_copy(posz_hbm, sposz)
        pltpu.sync_copy(d2e_hbm, sd2e)

    plsc.subcore_barrier()

    @pl.loop(0, S_GMAX)
    def _(g):
        chunk = wid + g * NWORK

        @pl.when(chunk < S_NCHUNK)
        def _():
            sl = pl.ds(chunk * CHUNK, CHUNK)
            pltpu.sync_copy(i_hbm.at[sl], bi)
            pltpu.sync_copy(j_hbm.at[sl], bj)
            pltpu.sync_copy(k_hbm.at[sl], bk)
            pltpu.sync_copy(e_hbm.at[sl], be)

            pltpu.sync_copy(sd2e.at[be], gd2)

            pltpu.sync_copy(sposx.at[bi], gxi)
            pltpu.sync_copy(sposy.at[bi], gyi)
            pltpu.sync_copy(sposz.at[bi], gzi)
            pltpu.sync_copy(sposx.at[bj], gxj)
            pltpu.sync_copy(sposy.at[bj], gyj)
            pltpu.sync_copy(sposz.at[bj], gzj)
            pltpu.sync_copy(sposx.at[bk], gxk)
            pltpu.sync_copy(sposy.at[bk], gyk)
            pltpu.sync_copy(sposz.at[bk], gzk)

            @pl.loop(0, CHUNK, step=LANES)
            def _(v):
                s = pl.ds(v, LANES)
                xi, yi, zi = gxi[s], gyi[s], gzi[s]
                r1x = gxj[s] - xi
                r1y = gyj[s] - yi
                r1z = gzj[s] - zi
                r2x = gxk[s] - xi
                r2y = gyk[s] - yi
                r2z = gzk[s] - zi
                oxdot[s] = r1x * r2x + r1y * r2y + r1z * r2z
                cx = r1y * r2z - r1z * r2y
                cy = r1z * r2x - r1x * r2z
                cz = r1x * r2y - r1y * r2x
                oy2[s] = cx * cx + cy * cy + cz * cz

            pltpu.sync_copy(oxdot, xdot_hbm.at[sl])
            pltpu.sync_copy(oy2, y2_hbm.at[sl])
            pltpu.sync_copy(gd2, d2_hbm.at[sl])


def _make_kerns():
    fvec = jax.ShapeDtypeStruct((SLEN,), jnp.float32)
    f32 = jnp.float32
    i32 = jnp.int32
    mesh = plsc.VectorSubcoreMesh(core_axis_name="c", subcore_axis_name="s")

    edge_kern = pl.kernel(
        _edge_body,
        out_type=jax.ShapeDtypeStruct((N_EDGES,), f32),
        mesh=mesh,
        scratch_types=[
            pltpu.VMEM_SHARED((N_NODES,), f32),
            pltpu.VMEM_SHARED((N_NODES,), f32),
            pltpu.VMEM_SHARED((N_NODES,), f32),
            pltpu.VMEM((CHUNK,), i32),
            pltpu.VMEM((CHUNK,), i32),
        ] + [pltpu.VMEM((CHUNK,), f32) for _ in range(7)],
    )

    trip_kern = pl.kernel(
        _triplet_body,
        out_type=[fvec, fvec, fvec],
        mesh=mesh,
        scratch_types=[
            pltpu.VMEM_SHARED((N_NODES,), f32),
            pltpu.VMEM_SHARED((N_NODES,), f32),
            pltpu.VMEM_SHARED((N_NODES,), f32),
            pltpu.VMEM_SHARED((N_EDGES,), f32),
            pltpu.VMEM((CHUNK,), i32),
            pltpu.VMEM((CHUNK,), i32),
            pltpu.VMEM((CHUNK,), i32),
            pltpu.VMEM((CHUNK,), i32),
        ] + [pltpu.VMEM((CHUNK,), f32) for _ in range(12)],
    )
    return edge_kern, trip_kern


def _basis_body(x_ref, y2_ref, d2_ref, w_ref, o_ref, sbf_ref):
    x = x_ref[0]
    y2 = y2_ref[0]
    d2 = d2_ref[0]

    yv = jnp.maximum(jnp.sqrt(y2), 1e-6)
    ca = x * lax.rsqrt(x * x + yv * yv)
    d = jnp.maximum(jnp.sqrt(d2), 1e-6)
    ds = d * (1.0 / CUTOFF)
    env = jnp.where(ds <= 1.0, 0.5 * jnp.cos(math.pi * ds) + 0.5, 0.0)
    inv = env / jnp.maximum(ds, 1e-6)

    rbf = [inv * jnp.sin(((r + 1) * math.pi) * ds) for r in range(NUM_RADIAL)]

    c2 = ca * ca
    c3 = c2 * ca
    c4 = c2 * c2
    half_rpi = 0.5 / math.sqrt(math.pi)
    cbf = [
        jnp.full_like(ca, half_rpi),
        math.sqrt(3.0 / (4.0 * math.pi)) * ca,
        math.sqrt(5.0 / (4.0 * math.pi)) * ((3.0 * c2 - 1.0) * 0.5),
        math.sqrt(7.0 / (4.0 * math.pi)) * ((5.0 * c3 - 3.0 * ca) * 0.5),
        math.sqrt(9.0 / (4.0 * math.pi)) * ((35.0 * c4 - 30.0 * c2 + 3.0) * 0.125),
        math.sqrt(11.0 / (4.0 * math.pi)) * ((63.0 * c3 * c2 - 70.0 * c3 + 15.0 * ca) * 0.125),
        math.sqrt(13.0 / (4.0 * math.pi)) * (32.0 * c3 * c3 - 48.0 * c4 + 18.0 * c2 - 1.0),
    ]

    for l in range(NUM_SPHERICAL):
        for r in range(NUM_RADIAL):
            sbf_ref[pl.ds(l * NUM_RADIAL + r, 1), :] = cbf[l] * rbf[r]

    o_ref[...] = lax.dot_general(
        sbf_ref[...], w_ref[...], (((0,), (0,)), ((), ())),
        preferred_element_type=jnp.float32)


def _basis_body_acc(x_ref, y2_ref, d2_ref, w_ref, acc_ref, o_ref, sbf_ref):
    del acc_ref
    _basis_body(x_ref, y2_ref, d2_ref, w_ref, o_ref, sbf_ref)


def _basis_project_slice(xdot, y2, d2, W_sbf, s, acc):
    in_spec = pl.BlockSpec((1, 1, BT), lambda i: (i, 0, 0))
    w_spec = pl.BlockSpec((NUM_SPHERICAL * NUM_RADIAL, HIDDEN),
                          lambda i: (0, 0))
    out_spec = pl.BlockSpec((BT, HIDDEN), lambda i, _s=s: (_s * NBS + i, 0))
    out_shape = jax.ShapeDtypeStruct((N_TRIPLETS, HIDDEN), jnp.float32)
    scratch = [pltpu.VMEM((NUM_SPHERICAL * NUM_RADIAL, BT), jnp.float32)]
    args = (xdot.reshape(NBS, 1, BT), y2.reshape(NBS, 1, BT),
            d2.reshape(NBS, 1, BT), W_sbf)
    if acc is None:
        return pl.pallas_call(
            _basis_body,
            grid=(NBS,),
            in_specs=[in_spec, in_spec, in_spec, w_spec],
            out_specs=out_spec,
            out_shape=out_shape,
            scratch_shapes=scratch,
        )(*args)
    return pl.pallas_call(
        _basis_body_acc,
        grid=(NBS,),
        in_specs=[in_spec, in_spec, in_spec, w_spec,
                  pl.BlockSpec(memory_space=pl.ANY)],
        out_specs=out_spec,
        out_shape=out_shape,
        scratch_shapes=scratch,
        input_output_aliases={4: 0},
    )(*args, acc)


def kernel(pos, edge_index, id3_i, id3_j, id3_k, id_expand_kj, W_sbf):
    posx = jnp.asarray(pos[:, 0])
    posy = jnp.asarray(pos[:, 1])
    posz = jnp.asarray(pos[:, 2])
    ei0 = edge_index[0]
    ei1 = edge_index[1]

    edge_kern, trip_kern = _make_kerns()
    d2e = edge_kern(ei0, ei1, posx, posy, posz)

    out = None
    for s in range(SLICES):
        sl = slice(s * SLEN, (s + 1) * SLEN)
        xdot, y2, d2 = trip_kern(id3_i[sl], id3_j[sl], id3_k[sl],
                                 id_expand_kj[sl], posx, posy, posz, d2e)
        out = _basis_project_slice(xdot, y2, d2, W_sbf, s, out)
    return out


# speedup vs baseline: 1.3194x; 1.3194x over previous
import math

import jax
import jax.numpy as jnp
from jax import lax
from jax.experimental import pallas as pl
from jax.experimental.pallas import tpu as pltpu
from jax.experimental.pallas import tpu_sc as plsc

N_NODES = 50000
N_EDGES = 800000
N_TRIPLETS = 800000
NUM_RADIAL = 6
NUM_SPHERICAL = 7
HIDDEN = 128
CUTOFF = 5.0

SC_CORES = 2
SC_SUBCORES = 16
NWORK = SC_CORES * SC_SUBCORES
CHUNK = 2000
NCHUNK = N_TRIPLETS // CHUNK
GMAX = (NCHUNK + NWORK - 1) // NWORK
LANES = 16

BT = 16000
NBT = N_TRIPLETS // BT


def _drain(handles):
    for h in handles:
        h.wait()


def _edge_body(ei0_hbm, ei1_hbm, posx_hbm, posy_hbm, posz_hbm,
               d2e_hbm,
               sposx, sposy, sposz,
               ba0, ba1, gxa, gya, gza, gxb, gyb, gzb, od2, sem):
    cid = lax.axis_index("c")
    sid = lax.axis_index("s")
    wid = sid * SC_CORES + cid

    @pl.when(sid == 0)
    def _():
        pltpu.sync_copy(posx_hbm, sposx)
        pltpu.sync_copy(posy_hbm, sposy)
        pltpu.sync_copy(posz_hbm, sposz)

    plsc.subcore_barrier()

    @pl.loop(0, GMAX)
    def _(g):
        chunk = wid + g * NWORK

        @pl.when(chunk < NCHUNK)
        def _():
            sl = pl.ds(chunk * CHUNK, CHUNK)
            _drain([pltpu.async_copy(ei0_hbm.at[sl], ba0, sem),
                    pltpu.async_copy(ei1_hbm.at[sl], ba1, sem)])
            _drain([pltpu.async_copy(sposx.at[ba0], gxa, sem),
                    pltpu.async_copy(sposy.at[ba0], gya, sem),
                    pltpu.async_copy(sposz.at[ba0], gza, sem),
                    pltpu.async_copy(sposx.at[ba1], gxb, sem),
                    pltpu.async_copy(sposy.at[ba1], gyb, sem),
                    pltpu.async_copy(sposz.at[ba1], gzb, sem)])

            @pl.loop(0, CHUNK, step=LANES)
            def _(v):
                s = pl.ds(v, LANES)
                dx = gxa[s] - gxb[s]
                dy = gya[s] - gyb[s]
                dz = gza[s] - gzb[s]
                od2[s] = dx * dx + dy * dy + dz * dz

            pltpu.sync_copy(od2, d2e_hbm.at[sl])


def _triplet_body(i_hbm, j_hbm, k_hbm, e_hbm,
                  posx_hbm, posy_hbm, posz_hbm, d2e_hbm,
                  xdot_hbm, y2_hbm, d2_hbm,
                  sposx, sposy, sposz, sd2e,
                  bi, bj, bk, be,
                  gxi, gyi, gzi, gxj, gyj, gzj, gxk, gyk, gzk,
                  gd2, oxdot, oy2, sem):
    cid = lax.axis_index("c")
    sid = lax.axis_index("s")
    wid = sid * SC_CORES + cid

    @pl.when(sid == 0)
    def _():
        pltpu.sync_copy(posx_hbm, sposx)
        pltpu.sync_copy(posy_hbm, sposy)
        pltpu.sync_copy(posz_hbm, sposz)
        pltpu.sync_copy(d2e_hbm, sd2e)

    plsc.subcore_barrier()

    @pl.loop(0, GMAX)
    def _(g):
        chunk = wid + g * NWORK

        @pl.when(chunk < NCHUNK)
        def _():
            sl = pl.ds(chunk * CHUNK, CHUNK)
            _drain([pltpu.async_copy(i_hbm.at[sl], bi, sem),
                    pltpu.async_copy(j_hbm.at[sl], bj, sem),
                    pltpu.async_copy(k_hbm.at[sl], bk, sem),
                    pltpu.async_copy(e_hbm.at[sl], be, sem)])
            _drain([pltpu.async_copy(sd2e.at[be], gd2, sem),
                    pltpu.async_copy(sposx.at[bi], gxi, sem),
                    pltpu.async_copy(sposy.at[bi], gyi, sem),
                    pltpu.async_copy(sposz.at[bi], gzi, sem),
                    pltpu.async_copy(sposx.at[bj], gxj, sem),
                    pltpu.async_copy(sposy.at[bj], gyj, sem),
                    pltpu.async_copy(sposz.at[bj], gzj, sem),
                    pltpu.async_copy(sposx.at[bk], gxk, sem),
                    pltpu.async_copy(sposy.at[bk], gyk, sem),
                    pltpu.async_copy(sposz.at[bk], gzk, sem)])

            @pl.loop(0, CHUNK, step=LANES)
            def _(v):
                s = pl.ds(v, LANES)
                xi, yi, zi = gxi[s], gyi[s], gzi[s]
                r1x = gxj[s] - xi
                r1y = gyj[s] - yi
                r1z = gzj[s] - zi
                r2x = gxk[s] - xi
                r2y = gyk[s] - yi
                r2z = gzk[s] - zi
                oxdot[s] = r1x * r2x + r1y * r2y + r1z * r2z
                cx = r1y * r2z - r1z * r2y
                cy = r1z * r2x - r1x * r2z
                cz = r1x * r2y - r1y * r2x
                oy2[s] = cx * cx + cy * cy + cz * cz

            _drain([pltpu.async_copy(oxdot, xdot_hbm.at[sl], sem),
                    pltpu.async_copy(oy2, y2_hbm.at[sl], sem),
                    pltpu.async_copy(gd2, d2_hbm.at[sl], sem)])


def _geom(id3_i, id3_j, id3_k, id_expand_kj, posx, posy, posz, ei0, ei1):
    fvec = jax.ShapeDtypeStruct((N_TRIPLETS,), jnp.float32)
    f32 = jnp.float32
    i32 = jnp.int32
    mesh = plsc.VectorSubcoreMesh(core_axis_name="c", subcore_axis_name="s")

    edge_kern = pl.kernel(
        _edge_body,
        out_type=jax.ShapeDtypeStruct((N_EDGES,), f32),
        mesh=mesh,
        scratch_types=[
            pltpu.VMEM_SHARED((N_NODES,), f32),
            pltpu.VMEM_SHARED((N_NODES,), f32),
            pltpu.VMEM_SHARED((N_NODES,), f32),
            pltpu.VMEM((CHUNK,), i32),
            pltpu.VMEM((CHUNK,), i32),
        ] + [pltpu.VMEM((CHUNK,), f32) for _ in range(7)]
          + [pltpu.SemaphoreType.DMA],
    )
    d2e = edge_kern(ei0, ei1, posx, posy, posz)

    trip_kern = pl.kernel(
        _triplet_body,
        out_type=[fvec, fvec, fvec],
        mesh=mesh,
        scratch_types=[
            pltpu.VMEM_SHARED((N_NODES,), f32),
            pltpu.VMEM_SHARED((N_NODES,), f32),
            pltpu.VMEM_SHARED((N_NODES,), f32),
            pltpu.VMEM_SHARED((N_EDGES,), f32),
            pltpu.VMEM((CHUNK,), i32),
            pltpu.VMEM((CHUNK,), i32),
            pltpu.VMEM((CHUNK,), i32),
            pltpu.VMEM((CHUNK,), i32),
        ] + [pltpu.VMEM((CHUNK,), f32) for _ in range(12)]
          + [pltpu.SemaphoreType.DMA],
    )
    return trip_kern(id3_i, id3_j, id3_k, id_expand_kj,
                     posx, posy, posz, d2e)


def _basis_body(x_ref, y2_ref, d2_ref, w_ref, o_ref, sbf_ref):
    x = x_ref[0]
    y2 = y2_ref[0]
    d2 = d2_ref[0]

    yv = jnp.maximum(jnp.sqrt(y2), 1e-6)
    ca = x * lax.rsqrt(x * x + yv * yv)
    d = jnp.maximum(jnp.sqrt(d2), 1e-6)
    ds = d * (1.0 / CUTOFF)
    env = jnp.where(ds <= 1.0, 0.5 * jnp.cos(math.pi * ds) + 0.5, 0.0)
    inv = env / jnp.maximum(ds, 1e-6)

    rbf = [inv * jnp.sin(((r + 1) * math.pi) * ds) for r in range(NUM_RADIAL)]

    c2 = ca * ca
    c3 = c2 * ca
    c4 = c2 * c2
    half_rpi = 0.5 / math.sqrt(math.pi)
    cbf = [
        jnp.full_like(ca, half_rpi),
        math.sqrt(3.0 / (4.0 * math.pi)) * ca,
        math.sqrt(5.0 / (4.0 * math.pi)) * ((3.0 * c2 - 1.0) * 0.5),
        math.sqrt(7.0 / (4.0 * math.pi)) * ((5.0 * c3 - 3.0 * ca) * 0.5),
        math.sqrt(9.0 / (4.0 * math.pi)) * ((35.0 * c4 - 30.0 * c2 + 3.0) * 0.125),
        math.sqrt(11.0 / (4.0 * math.pi)) * ((63.0 * c3 * c2 - 70.0 * c3 + 15.0 * ca) * 0.125),
        math.sqrt(13.0 / (4.0 * math.pi)) * (32.0 * c3 * c3 - 48.0 * c4 + 18.0 * c2 - 1.0),
    ]

    for l in range(NUM_SPHERICAL):
        for r in range(NUM_RADIAL):
            sbf_ref[pl.ds(l * NUM_RADIAL + r, 1), :] = cbf[l] * rbf[r]

    o_ref[...] = lax.dot_general(
        sbf_ref[...], w_ref[...], (((0,), (0,)), ((), ())),
        preferred_element_type=jnp.float32)


def _basis_project(xdot, y2, d2, W_sbf):
    in_spec = pl.BlockSpec((1, 1, BT), lambda i: (i, 0, 0))
    return pl.pallas_call(
        _basis_body,
        grid=(NBT,),
        in_specs=[in_spec, in_spec, in_spec,
                  pl.BlockSpec((NUM_SPHERICAL * NUM_RADIAL, HIDDEN),
                               lambda i: (0, 0))],
        out_specs=pl.BlockSpec((BT, HIDDEN), lambda i: (i, 0)),
        out_shape=jax.ShapeDtypeStruct((N_TRIPLETS, HIDDEN), jnp.float32),
        scratch_shapes=[pltpu.VMEM((NUM_SPHERICAL * NUM_RADIAL, BT), jnp.float32)],
    )(xdot.reshape(NBT, 1, BT), y2.reshape(NBT, 1, BT),
      d2.reshape(NBT, 1, BT), W_sbf)


def kernel(pos, edge_index, id3_i, id3_j, id3_k, id_expand_kj, W_sbf):
    posx = jnp.asarray(pos[:, 0])
    posy = jnp.asarray(pos[:, 1])
    posz = jnp.asarray(pos[:, 2])
    ei0 = edge_index[0]
    ei1 = edge_index[1]
    xdot, y2, d2 = _geom(id3_i, id3_j, id3_k, id_expand_kj,
                         posx, posy, posz, ei0, ei1)
    return _basis_project(xdot, y2, d2, W_sbf)
